# baseline (speedup 1.0000x reference)
import jax
import jax.numpy as jnp
from jax.experimental import pallas as pl

_B, _N, _D_FEAT, _HID, _LAYERS = 8, 256, 256, 256, 3
_N_TYPES, _N_CLASSES = 10, 8
_GB = 4
_GRID = _B // _GB


def _dgt(x, w):
    return jax.lax.dot_general(x, w, (((1,), (1,)), ((), ())),
                               preferred_element_type=jnp.float32)


def _gnn_body(nf_ref, adj_ref, nt_ref, emb_ref, projw_ref, projb_ref,
              linw_ref, linb_ref, attw_ref, attb_ref,
              poolw1_ref, poolb1_ref, poolw2_ref,
              clsw1_ref, clsb1_ref, clsw2_ref, clsb2_ref,
              scores_ref, gemb_ref):
    f32 = jnp.float32
    x = nf_ref[...].reshape(_GB * _N, _D_FEAT)
    feat = _dgt(x, projw_ref[...]) + projb_ref[...].reshape(1, _HID)
    kiota = jax.lax.broadcasted_iota(jnp.int32, (_N_TYPES, _N), 0)
    hb_list = []
    for b in range(_GB):
        ohT = (kiota == nt_ref[0, b:b + 1, :]).astype(f32)
        te = jax.lax.dot_general(ohT, emb_ref[...], (((0,), (0,)), ((), ())),
                                 preferred_element_type=f32)
        hb_list.append(feat[b * _N:(b + 1) * _N] + te)
    h = jnp.concatenate(hb_list, axis=0)
    mask = (adj_ref[0] > 0.0).astype(f32)
    for l in range(_LAYERS):
        t = _dgt(h, linw_ref[l]) + linb_ref[l:l + 1]
        aw = attw_ref[l]
        w1 = aw[:, :_HID]
        w2 = aw[:, _HID:]
        s1 = _dgt(t, w1)
        blocks = []
        for b in range(_GB):
            tb = t[b * _N:(b + 1) * _N]
            s2 = _dgt(w2, tb)
            logits = s1[b * _N:(b + 1) * _N] + s2 + attb_ref[l:l + 1]
            w = jax.nn.sigmoid(logits) * mask
            blocks.append(jax.nn.relu(tb + jnp.dot(w, tb, preferred_element_type=f32)))
        h = jnp.concatenate(blocks, axis=0)
    hp = jnp.tanh(_dgt(h, poolw1_ref[...]) + poolb1_ref[...].reshape(1, _HID // 2))
    a = _dgt(hp, poolw2_ref[...])
    gs = []
    for b in range(_GB):
        ab = jax.nn.softmax(a[b * _N:(b + 1) * _N], axis=0)
        hb = h[b * _N:(b + 1) * _N]
        gs.append(jax.lax.dot_general(ab, hb, (((0,), (0,)), ((), ())),
                                      preferred_element_type=f32))
    g = jnp.concatenate(gs, axis=0)
    z = jax.nn.relu(_dgt(g, clsw1_ref[...]) + clsb1_ref[...].reshape(1, _HID // 2))
    s = _dgt(z, clsw2_ref[...]) + clsb2_ref[...].reshape(1, _N_CLASSES)
    scores_ref[0] = s
    gemb_ref[0] = g


def kernel(node_features, adjacency, node_types, emb_table, proj_w, proj_b,
           lin_w, lin_b, att_w, att_b, pool_w1, pool_b1, pool_w2, pool_b2,
           cls_w1, cls_b1, cls_w2, cls_b2):
    f32 = jnp.float32
    del pool_b2
    nt3 = node_types.astype(jnp.int32).reshape(_GRID, _GB, _N)

    def full(shape):
        n = len(shape)
        return pl.BlockSpec(shape, lambda i, _n=n: (0,) * _n)

    scores, gemb = pl.pallas_call(
        _gnn_body,
        grid=(_GRID,),
        in_specs=[
            pl.BlockSpec((_GB, _N, _D_FEAT), lambda i: (i, 0, 0)),
            pl.BlockSpec((1, _N, _N), lambda i: (0, 0, 0)),
            pl.BlockSpec((1, _GB, _N), lambda i: (i, 0, 0)),
            full((_N_TYPES, _HID)),
            full((_HID, _D_FEAT)),
            full((_HID,)),
            full((_LAYERS, _HID, _HID)),
            full((_LAYERS, _HID)),
            full((_LAYERS, 1, 2 * _HID)),
            full((_LAYERS, 1)),
            full((_HID // 2, _HID)),
            full((_HID // 2,)),
            full((1, _HID // 2)),
            full((_HID // 2, _HID)),
            full((_HID // 2,)),
            full((_N_CLASSES, _HID // 2)),
            full((_N_CLASSES,)),
        ],
        out_specs=[
            pl.BlockSpec((1, _GB, _N_CLASSES), lambda i: (i, 0, 0)),
            pl.BlockSpec((1, _GB, _HID), lambda i: (i, 0, 0)),
        ],
        out_shape=[
            jax.ShapeDtypeStruct((_GRID, _GB, _N_CLASSES), f32),
            jax.ShapeDtypeStruct((_GRID, _GB, _HID), f32),
        ],
    )(node_features, adjacency, nt3, emb_table,
      proj_w, proj_b, lin_w, lin_b, att_w, att_b,
      pool_w1, pool_b1, pool_w2, cls_w1, cls_b1, cls_w2, cls_b2)
    return scores.reshape(_B, _N_CLASSES), gemb.reshape(_B, _HID)


# speedup vs baseline: 1.2940x; 1.2940x over previous
import jax
import jax.numpy as jnp
from jax.experimental import pallas as pl
from jax.experimental.pallas import tpu as pltpu

_B, _N, _D_FEAT, _HID, _LAYERS = 8, 256, 256, 256, 3
_N_TYPES, _N_CLASSES = 10, 8


def _dgt(x, w):
    return jax.lax.dot_general(x, w, (((1,), (1,)), ((), ())),
                               preferred_element_type=jnp.float32)


def _gnn_body(nf_ref, adj_ref, nt_ref, emb_ref, projw_ref, projb_ref,
              linw_hbm, linb_ref, attw_ref, attb_ref,
              poolw1_hbm, poolb1_ref, poolw2_ref,
              clsw1_hbm, clsb1_ref, clsw2_ref, clsb2_ref,
              scores_ref, gemb_ref,
              linw_vm, poolw1_vm, clsw1_vm, lin_sem, tail_sem, cls_sem):
    f32 = jnp.float32
    lin_cp = pltpu.make_async_copy(linw_hbm, linw_vm, lin_sem)
    lin_cp.start()
    tail_cp = pltpu.make_async_copy(poolw1_hbm, poolw1_vm, tail_sem)
    tail_cp.start()
    cls_cp = pltpu.make_async_copy(clsw1_hbm, clsw1_vm, cls_sem)
    cls_cp.start()
    x = nf_ref[...].reshape(_B * _N, _D_FEAT)
    feat = _dgt(x, projw_ref[...]) + projb_ref[...].reshape(1, _HID)
    kiota = jax.lax.broadcasted_iota(jnp.int32, (_N_TYPES, _N), 0)
    hb_list = []
    for b in range(_B):
        ohT = (kiota == nt_ref[b:b + 1, :]).astype(f32)
        te = jax.lax.dot_general(ohT, emb_ref[...], (((0,), (0,)), ((), ())),
                                 preferred_element_type=f32)
        hb_list.append(feat[b * _N:(b + 1) * _N] + te)
    h = jnp.concatenate(hb_list, axis=0)
    mask = (adj_ref[0] > 0.0).astype(f32)
    lin_cp.wait()
    for l in range(_LAYERS):
        t = _dgt(h, linw_vm[l]) + linb_ref[l:l + 1]
        aw = attw_ref[l]
        w1 = aw[:, :_HID]
        w2 = aw[:, _HID:]
        s1 = _dgt(t, w1)
        blocks = []
        for b in range(_B):
            tb = t[b * _N:(b + 1) * _N]
            s2 = _dgt(w2, tb)
            logits = s1[b * _N:(b + 1) * _N] + s2 + attb_ref[l:l + 1]
            w = (0.5 * jnp.tanh(0.5 * logits) + 0.5) * mask
            blocks.append(jax.nn.relu(tb + jnp.dot(w, tb, preferred_element_type=f32)))
        h = jnp.concatenate(blocks, axis=0)
    tail_cp.wait()
    cls_cp.wait()
    hp = jnp.tanh(_dgt(h, poolw1_vm[...]) + poolb1_ref[...].reshape(1, _HID // 2))
    a = _dgt(hp, poolw2_ref[...])
    gs = []
    for b in range(_B):
        ab = jax.nn.softmax(a[b * _N:(b + 1) * _N], axis=0)
        hb = h[b * _N:(b + 1) * _N]
        gs.append(jax.lax.dot_general(ab, hb, (((0,), (0,)), ((), ())),
                                      preferred_element_type=f32))
    g = jnp.concatenate(gs, axis=0)
    z = jax.nn.relu(_dgt(g, clsw1_vm[...]) + clsb1_ref[...].reshape(1, _HID // 2))
    scores_ref[...] = _dgt(z, clsw2_ref[...]) + clsb2_ref[...].reshape(1, _N_CLASSES)
    gemb_ref[...] = g


def kernel(node_features, adjacency, node_types, emb_table, proj_w, proj_b,
           lin_w, lin_b, att_w, att_b, pool_w1, pool_b1, pool_w2, pool_b2,
           cls_w1, cls_b1, cls_w2, cls_b2):
    f32 = jnp.float32
    del pool_b2

    def full(shape):
        n = len(shape)
        return pl.BlockSpec(shape, lambda i, _n=n: (0,) * _n)

    scores, gemb = pl.pallas_call(
        _gnn_body,
        grid=(1,),
        in_specs=[
            full((_B, _N, _D_FEAT)),
            pl.BlockSpec((1, _N, _N), lambda i: (0, 0, 0)),
            full((_B, _N)),
            full((_N_TYPES, _HID)),
            full((_HID, _D_FEAT)),
            full((_HID,)),
            pl.BlockSpec(memory_space=pl.ANY),
            full((_LAYERS, _HID)),
            full((_LAYERS, 1, 2 * _HID)),
            full((_LAYERS, 1)),
            pl.BlockSpec(memory_space=pl.ANY),
            full((_HID // 2,)),
            full((1, _HID // 2)),
            pl.BlockSpec(memory_space=pl.ANY),
            full((_HID // 2,)),
            full((_N_CLASSES, _HID // 2)),
            full((_N_CLASSES,)),
        ],
        scratch_shapes=[
            pltpu.VMEM((_LAYERS, _HID, _HID), f32),
            pltpu.VMEM((_HID // 2, _HID), f32),
            pltpu.VMEM((_HID // 2, _HID), f32),
            pltpu.SemaphoreType.DMA,
            pltpu.SemaphoreType.DMA,
            pltpu.SemaphoreType.DMA,
        ],
        out_specs=[
            full((_B, _N_CLASSES)),
            full((_B, _HID)),
        ],
        out_shape=[
            jax.ShapeDtypeStruct((_B, _N_CLASSES), f32),
            jax.ShapeDtypeStruct((_B, _HID), f32),
        ],
    )(node_features, adjacency, node_types.astype(jnp.int32), emb_table,
      proj_w, proj_b, lin_w, lin_b, att_w, att_b,
      pool_w1, pool_b1, pool_w2, cls_w1, cls_b1, cls_w2, cls_b2)
    return scores, gemb


# speedup vs baseline: 1.3170x; 1.0177x over previous
import jax
import jax.numpy as jnp
from jax.experimental import pallas as pl

_B, _N, _D_FEAT, _HID, _LAYERS = 8, 256, 256, 256, 3
_N_TYPES, _N_CLASSES = 10, 8


def _dgt(x, w):
    return jax.lax.dot_general(x, w, (((1,), (1,)), ((), ())),
                               preferred_element_type=jnp.float32)


def _gnn_body(nf_ref, adj_ref, nt_ref, emb_ref, projw_ref, projb_ref,
              linw_ref, linb_ref, attw_ref, attb_ref,
              poolw1_ref, poolb1_ref, poolw2_ref,
              clsw1_ref, clsb1_ref, clsw2_ref, clsb2_ref,
              scores_ref, gemb_ref):
    f32 = jnp.float32
    x = nf_ref[...].reshape(_B * _N, _D_FEAT)
    feat = _dgt(x, projw_ref[...]) + projb_ref[...].reshape(1, _HID)
    kiota = jax.lax.broadcasted_iota(jnp.int32, (_N_TYPES, _N), 0)
    hb_list = []
    for b in range(_B):
        ohT = (kiota == nt_ref[b:b + 1, :]).astype(f32)
        te = jax.lax.dot_general(ohT, emb_ref[...], (((0,), (0,)), ((), ())),
                                 preferred_element_type=f32)
        hb_list.append(feat[b * _N:(b + 1) * _N] + te)
    h = jnp.concatenate(hb_list, axis=0)
    mask = (adj_ref[0] > 0.0).astype(f32)
    for l in range(_LAYERS):
        t = _dgt(h, linw_ref[l]) + linb_ref[l:l + 1]
        aw = attw_ref[l]
        w1 = aw[:, :_HID]
        w2 = aw[:, _HID:]
        s1 = _dgt(t, w1)
        blocks = []
        for b in range(_B):
            tb = t[b * _N:(b + 1) * _N]
            s2 = _dgt(w2, tb)
            logits = s1[b * _N:(b + 1) * _N] + s2 + attb_ref[l:l + 1]
            w = (0.5 * jnp.tanh(0.5 * logits) + 0.5) * mask
            blocks.append(jax.nn.relu(tb + jnp.dot(w, tb, preferred_element_type=f32)))
        h = jnp.concatenate(blocks, axis=0)
    hp = jnp.tanh(_dgt(h, poolw1_ref[...]) + poolb1_ref[...].reshape(1, _HID // 2))
    a = _dgt(hp, poolw2_ref[...])
    gs = []
    for b in range(_B):
        ab = jax.nn.softmax(a[b * _N:(b + 1) * _N], axis=0)
        hb = h[b * _N:(b + 1) * _N]
        gs.append(jax.lax.dot_general(ab, hb, (((0,), (0,)), ((), ())),
                                      preferred_element_type=f32))
    g = jnp.concatenate(gs, axis=0)
    z = jax.nn.relu(_dgt(g, clsw1_ref[...]) + clsb1_ref[...].reshape(1, _HID // 2))
    scores_ref[...] = _dgt(z, clsw2_ref[...]) + clsb2_ref[...].reshape(1, _N_CLASSES)
    gemb_ref[...] = g


def kernel(node_features, adjacency, node_types, emb_table, proj_w, proj_b,
           lin_w, lin_b, att_w, att_b, pool_w1, pool_b1, pool_w2, pool_b2,
           cls_w1, cls_b1, cls_w2, cls_b2):
    f32 = jnp.float32
    del pool_b2

    def full(shape):
        n = len(shape)
        return pl.BlockSpec(shape, lambda i, _n=n: (0,) * _n)

    scores, gemb = pl.pallas_call(
        _gnn_body,
        grid=(1,),
        in_specs=[
            full((_B, _N, _D_FEAT)),
            pl.BlockSpec((1, _N, _N), lambda i: (0, 0, 0)),
            full((_B, _N)),
            full((_N_TYPES, _HID)),
            full((_HID, _D_FEAT)),
            full((_HID,)),
            full((_LAYERS, _HID, _HID)),
            full((_LAYERS, _HID)),
            full((_LAYERS, 1, 2 * _HID)),
            full((_LAYERS, 1)),
            full((_HID // 2, _HID)),
            full((_HID // 2,)),
            full((1, _HID // 2)),
            full((_HID // 2, _HID)),
            full((_HID // 2,)),
            full((_N_CLASSES, _HID // 2)),
            full((_N_CLASSES,)),
        ],
        out_specs=[
            full((_B, _N_CLASSES)),
            full((_B, _HID)),
        ],
        out_shape=[
            jax.ShapeDtypeStruct((_B, _N_CLASSES), f32),
            jax.ShapeDtypeStruct((_B, _HID), f32),
        ],
    )(node_features, adjacency, node_types.astype(jnp.int32), emb_table,
      proj_w, proj_b, lin_w, lin_b, att_w, att_b,
      pool_w1, pool_b1, pool_w2, cls_w1, cls_b1, cls_w2, cls_b2)
    return scores, gemb
